# Initial kernel scaffold; baseline (speedup 1.0000x reference)
import jax
import jax.numpy as jnp
from jax.experimental import pallas as pl


def kernel(features, edge_index, W_pre, b_pre, W1, b1, a1, W2, b2, a2, W_post, b_post):
    raise NotImplementedError("write your pallas kernel here")


# speedup vs baseline: 2.8268x; 2.8268x over previous
import functools

import jax
import jax.numpy as jnp
from jax import lax
from jax.experimental import pallas as pl
from jax.experimental.pallas import tpu as pltpu
from jax.experimental.pallas import tpu_sc as plsc

N = 10000
NPAD = 10240
IN_SIZE = 256
HID = 512
OUT_SIZE = 256
C = 4
FC = 128
E = 160000
TILES = 16
EB = 128
NB = 79
EPT = NB * EB
EPAD = TILES * EPT
RPT = NPAD // TILES

_MESH = plsc.VectorSubcoreMesh(core_axis_name="c", subcore_axis_name="s")


def _seg_body(h0, h1, h2, h3, srcr, dstr, zeros_hbm,
              o0, o1, o2, o3,
              src_v, dst_v, rows_v, acc, sem):
    cid = lax.axis_index("c")
    sid = lax.axis_index("s")
    row0 = sid * RPT

    pltpu.sync_copy(srcr.at[sid], src_v)
    pltpu.sync_copy(dstr.at[sid], dst_v)

    def process(hc, outc):
        pltpu.sync_copy(zeros_hbm.at[pl.ds(row0, RPT)], acc.at[pl.ds(row0, RPT)])
        plsc.subcore_barrier()

        def body(j, carry):
            pltpu.async_copy(hc.at[src_v.at[j]], rows_v, sem).wait()
            pltpu.sync_copy(rows_v, acc.at[dst_v.at[j]], add=True)
            return carry

        lax.fori_loop(0, NB, body, 0, unroll=False)
        plsc.subcore_barrier()
        pltpu.sync_copy(acc.at[pl.ds(row0, RPT)], outc.at[pl.ds(row0, RPT)])
        plsc.subcore_barrier()

    @pl.when(cid == 0)
    def _():
        process(h0, o0)
        process(h1, o1)

    @pl.when(cid == 1)
    def _():
        process(h2, o2)
        process(h3, o3)


@functools.partial(jax.jit, donate_argnums=())
def _segment_sum_sc(hc, srcr, dstr, zeros_hbm):
    out = pl.kernel(
        _seg_body,
        out_type=[jax.ShapeDtypeStruct((NPAD, FC), jnp.float32)] * C,
        mesh=_MESH,
        scratch_types=[
            pltpu.VMEM((NB, EB), jnp.int32),
            pltpu.VMEM((NB, EB), jnp.int32),
            pltpu.VMEM((EB, FC), jnp.float32),
            pltpu.VMEM_SHARED((NPAD, FC), jnp.float32),
            pltpu.SemaphoreType.DMA,
        ],
    )(hc[0], hc[1], hc[2], hc[3], srcr, dstr, zeros_hbm)
    return jnp.stack(out)


def _pre_body(x_ref, w_ref, b_ref, o_ref):
    acc = jnp.dot(x_ref[...], w_ref[...], preferred_element_type=jnp.float32)
    for c in range(C):
        o_ref[c] = jnp.maximum(acc[:, c * FC:(c + 1) * FC] + b_ref[c], 0.0)


def _mid_body(h_ref, g_ref, w_ref, b_ref, a_ref, o_ref):
    s = h_ref[...] + g_ref[...]
    acc = jnp.dot(s[0], w_ref[0:FC, :], preferred_element_type=jnp.float32)
    for c in range(1, C):
        acc += jnp.dot(s[c], w_ref[c * FC:(c + 1) * FC, :],
                       preferred_element_type=jnp.float32)
    a = a_ref[0, 0]
    for c in range(C):
        v = acc[:, c * FC:(c + 1) * FC] + b_ref[c]
        o_ref[c] = jnp.where(v >= 0, v, a * v)


def _post_body(h_ref, w_ref, b_ref, o_ref):
    acc = jnp.dot(h_ref[0], w_ref[0:FC, :], preferred_element_type=jnp.float32)
    for c in range(1, C):
        acc += jnp.dot(h_ref[c], w_ref[c * FC:(c + 1) * FC, :],
                       preferred_element_type=jnp.float32)
    o_ref[...] = acc + b_ref[...]


_BR = 512
_GRID = (NPAD // _BR,)


def _pre_gemm(x, w, b):
    return pl.pallas_call(
        _pre_body,
        grid=_GRID,
        in_specs=[
            pl.BlockSpec((_BR, IN_SIZE), lambda i: (i, 0)),
            pl.BlockSpec((IN_SIZE, HID), lambda i: (0, 0)),
            pl.BlockSpec((C, FC), lambda i: (0, 0)),
        ],
        out_specs=pl.BlockSpec((C, _BR, FC), lambda i: (0, i, 0)),
        out_shape=jax.ShapeDtypeStruct((C, NPAD, FC), jnp.float32),
    )(x, w, b)


def _mid_gemm(h, g, w, b, a):
    return pl.pallas_call(
        _mid_body,
        grid=_GRID,
        in_specs=[
            pl.BlockSpec((C, _BR, FC), lambda i: (0, i, 0)),
            pl.BlockSpec((C, _BR, FC), lambda i: (0, i, 0)),
            pl.BlockSpec((HID, HID), lambda i: (0, 0)),
            pl.BlockSpec((C, FC), lambda i: (0, 0)),
            pl.BlockSpec(memory_space=pltpu.SMEM),
        ],
        out_specs=pl.BlockSpec((C, _BR, FC), lambda i: (0, i, 0)),
        out_shape=jax.ShapeDtypeStruct((C, NPAD, FC), jnp.float32),
    )(h, g, w, b, a)


def _post_gemm(h, w, b):
    return pl.pallas_call(
        _post_body,
        grid=_GRID,
        in_specs=[
            pl.BlockSpec((C, _BR, FC), lambda i: (0, i, 0)),
            pl.BlockSpec((HID, OUT_SIZE), lambda i: (0, 0)),
            pl.BlockSpec((1, OUT_SIZE), lambda i: (0, 0)),
        ],
        out_specs=pl.BlockSpec((_BR, OUT_SIZE), lambda i: (i, 0)),
        out_shape=jax.ShapeDtypeStruct((NPAD, OUT_SIZE), jnp.float32),
    )(h, w, b)


def kernel(features, edge_index, W_pre, b_pre, W1, b1, a1, W2, b2, a2, W_post, b_post):
    x = jnp.pad(features, ((0, NPAD - N), (0, 0)))
    src = edge_index[0].astype(jnp.int32)
    dst = edge_index[1].astype(jnp.int32)
    srcr = jnp.pad(src, (0, EPAD - E)).reshape(TILES, NB, EB)
    dstr = jnp.pad(dst, (0, EPAD - E), constant_values=N).reshape(TILES, NB, EB)
    zeros_hbm = jnp.zeros((NPAD, FC), jnp.float32)

    b_pre_c = b_pre.reshape(C, FC)
    b1_c = b1.reshape(C, FC)
    b2_c = b2.reshape(C, FC)
    a1_s = a1.reshape(1, 1)
    a2_s = a2.reshape(1, 1)

    h = _pre_gemm(x, W_pre, b_pre_c)
    g = _segment_sum_sc(h, srcr, dstr, zeros_hbm)
    h = _mid_gemm(h, g, W1, b1_c, a1_s)
    g = _segment_sum_sc(h, srcr, dstr, zeros_hbm)
    h = _mid_gemm(h, g, W2, b2_c, a2_s)
    out = _post_gemm(h, W_post, b_post.reshape(1, OUT_SIZE))
    return out[:N]
